# baseline (speedup 1.0000x reference)
import functools

import jax
import jax.numpy as jnp
from jax import lax
from jax.experimental import pallas as pl
from jax.experimental.pallas import tpu as pltpu
from jax.experimental.pallas import tpu_sc as plsc

T, B, D = 2048, 4, 1024
NC, NS, L = 2, 16, 16
NW = NC * NS
T_SC = 512
T_TC = T - T_SC
RPW = T_SC // NW
VECS = D // L
BT = 512


def _tc_body(x_ref, pos_ref, out_ref):
    out_ref[...] = x_ref[...] + pos_ref[...][:, None, :]


def _sc_body(x_hbm, pos_hbm, out_hbm, x_v, pos_v):
    wid = lax.axis_index("s") * NC + lax.axis_index("c")
    src0 = T_TC + wid * RPW
    pltpu.sync_copy(x_hbm.at[pl.ds(src0, RPW)], x_v)
    pltpu.sync_copy(pos_hbm.at[pl.ds(src0, RPW)], pos_v)

    def row_body(t, carry):
        for j in range(VECS):
            p = pos_v[t, pl.ds(j * L, L)]
            for b in range(B):
                x_v[t, b, pl.ds(j * L, L)] = x_v[t, b, pl.ds(j * L, L)] + p
        return carry

    lax.fori_loop(0, RPW, row_body, 0)
    pltpu.sync_copy(x_v, out_hbm.at[pl.ds(wid * RPW, RPW)])


def kernel(x, pos_table):
    mesh = plsc.VectorSubcoreMesh(core_axis_name="c", subcore_axis_name="s")
    sc_k = functools.partial(
        pl.kernel,
        mesh=mesh,
        out_type=jax.ShapeDtypeStruct((T_SC, B, D), jnp.float32),
        scratch_types=[
            pltpu.VMEM((RPW, B, D), jnp.float32),
            pltpu.VMEM((RPW, D), jnp.float32),
        ],
    )(_sc_body)
    sc_out = sc_k(x, pos_table)

    tc_out = pl.pallas_call(
        _tc_body,
        grid=(T_TC // BT,),
        in_specs=[
            pl.BlockSpec((BT, B, D), lambda i: (i, 0, 0)),
            pl.BlockSpec((BT, D), lambda i: (i, 0)),
        ],
        out_specs=pl.BlockSpec((BT, B, D), lambda i: (i, 0, 0)),
        out_shape=jax.ShapeDtypeStruct((T_TC, B, D), x.dtype),
    )(x[:T_TC], pos_table[:T_TC])

    return jnp.concatenate([tc_out, sc_out], axis=0)
